# baseline (speedup 1.0000x reference)
import functools

import jax
import jax.numpy as jnp
from jax.experimental import pallas as pl
from jax.experimental.pallas import tpu as pltpu


def _point_mean_kernel(x_ref, w0_ref, b0_ref, w1_ref, b1_ref, o_ref, acc_ref,
                       *, inv_l, n_l):
    l = pl.program_id(1)

    @pl.when(l == 0)
    def _():
        acc_ref[...] = jnp.zeros_like(acc_ref)

    x = x_ref[0].astype(jnp.bfloat16)
    h1 = jnp.dot(w0_ref[...], x, preferred_element_type=jnp.float32)
    h1 = jnp.maximum(h1 + b0_ref[...], 0.0)
    h2 = jnp.dot(w1_ref[...], h1.astype(jnp.bfloat16),
                 preferred_element_type=jnp.float32)
    h2 = jnp.maximum(h2 + b1_ref[...], 0.0)
    ones = jnp.ones((1, h2.shape[1]), jnp.bfloat16)
    part = jax.lax.dot_general(ones, h2.astype(jnp.bfloat16),
                               (((1,), (1,)), ((), ())),
                               preferred_element_type=jnp.float32)
    acc_ref[...] += part

    @pl.when(l == n_l - 1)
    def _():
        o_ref[...] = (acc_ref[...] * inv_l).reshape(o_ref.shape)


def _head_kernel(m_ref, w0_ref, b0_ref, w1_ref, b1_ref, w2_ref, b2_ref, o_ref):
    g = jnp.dot(m_ref[...], w0_ref[...], preferred_element_type=jnp.float32)
    g = jnp.maximum(g + b0_ref[...], 0.0)
    g = jnp.dot(g, w1_ref[...], preferred_element_type=jnp.float32)
    g = jnp.maximum(g + b1_ref[...], 0.0)
    o_ref[...] = (jnp.dot(g, w2_ref[...], preferred_element_type=jnp.float32)
                  + b2_ref[...])


def _pick_tl(L):
    for tl in (16384, 8192, 4096, 2048, 1024, 512, 256, 128):
        if L % tl == 0:
            return tl
    return L


def kernel(point_cloud, lw0, lb0, lw1, lb1, gw0, gb0, gw1, gb1, gw2, gb2):
    B, C, L = point_cloud.shape
    H1 = lw0.shape[1]
    H2 = lw1.shape[1]

    tl = _pick_tl(L)
    n_l = L // tl

    w0t = lw0.T.astype(jnp.bfloat16)
    b0c = lb0.reshape(H1, 1)
    w1t = lw1.T.astype(jnp.bfloat16)
    b1c = lb1.reshape(H2, 1)

    means = pl.pallas_call(
        functools.partial(_point_mean_kernel, inv_l=1.0 / L, n_l=n_l),
        out_shape=jax.ShapeDtypeStruct((B, 1, H2), jnp.float32),
        grid=(B, n_l),
        in_specs=[
            pl.BlockSpec((1, C, tl), lambda b, l: (b, 0, l)),
            pl.BlockSpec((H1, C), lambda b, l: (0, 0)),
            pl.BlockSpec((H1, 1), lambda b, l: (0, 0)),
            pl.BlockSpec((H2, H1), lambda b, l: (0, 0)),
            pl.BlockSpec((H2, 1), lambda b, l: (0, 0)),
        ],
        out_specs=pl.BlockSpec((1, 1, H2), lambda b, l: (b, 0, 0)),
        scratch_shapes=[pltpu.VMEM((1, H2), jnp.float32)],
        compiler_params=pltpu.CompilerParams(
            dimension_semantics=("parallel", "arbitrary"),
            vmem_limit_bytes=96 * 1024 * 1024,
        ),
    )(point_cloud, w0t, b0c, w1t, b1c)

    m = means.reshape(B, H2)

    out = pl.pallas_call(
        _head_kernel,
        out_shape=jax.ShapeDtypeStruct((B, 1), jnp.float32),
    )(m, gw0, gb0.reshape(1, -1), gw1, gb1.reshape(1, -1),
      gw2, gb2.reshape(1, 1))
    return out


# speedup vs baseline: 1.2994x; 1.2994x over previous
import functools

import jax
import jax.numpy as jnp
from jax.experimental import pallas as pl
from jax.experimental.pallas import tpu as pltpu


def _point_mean_kernel(x_ref, w0_ref, w1_ref, o_ref, *, inv_l, n_c):
    bb, _, L = x_ref.shape
    tc = L // n_c
    for b in range(bb):
        acc = None
        for c in range(n_c):
            xb = x_ref[b, :, c * tc:(c + 1) * tc].astype(jnp.bfloat16)
            x_aug = jnp.concatenate(
                [xb, jnp.ones((2, tc), jnp.bfloat16)], axis=0)
            h1 = jnp.dot(w0_ref[...], x_aug,
                         preferred_element_type=jnp.float32)
            h1b = jnp.maximum(h1.astype(jnp.bfloat16), 0)
            h2 = jnp.dot(w1_ref[...], h1b,
                         preferred_element_type=jnp.float32)
            h2 = jnp.maximum(h2, 0.0)
            w = tc
            while w > 128:
                w //= 2
                h2 = h2[:, :w] + h2[:, w:]
            acc = h2 if acc is None else acc + h2
        ones = jnp.ones((1, 128), jnp.float32)
        row = jax.lax.dot_general(ones, acc * inv_l,
                                  (((1,), (1,)), ((), ())),
                                  preferred_element_type=jnp.float32)
        o_ref[b] = row


def _head_kernel(m_ref, w0_ref, b0_ref, w1_ref, b1_ref, w2_ref, b2_ref, o_ref):
    g = jnp.dot(m_ref[...], w0_ref[...], preferred_element_type=jnp.float32)
    g = jnp.maximum(g + b0_ref[...], 0.0)
    g = jnp.dot(g, w1_ref[...], preferred_element_type=jnp.float32)
    g = jnp.maximum(g + b1_ref[...], 0.0)
    o_ref[...] = (jnp.dot(g, w2_ref[...], preferred_element_type=jnp.float32)
                  + b2_ref[...])


def _pick_tc(L):
    for tc in (16384, 8192, 4096, 2048, 1024, 512, 256, 128):
        if L % tc == 0:
            return tc
    return L


def kernel(point_cloud, lw0, lb0, lw1, lb1, gw0, gb0, gw1, gb1, gw2, gb2):
    B, C, L = point_cloud.shape
    H1 = lw0.shape[1]
    H2 = lw1.shape[1]

    tc = _pick_tc(L)
    n_c = L // tc
    bb = 2 if B % 2 == 0 else 1

    def _hi_lo(b):
        hi = b.astype(jnp.bfloat16).astype(jnp.float32)
        return hi, b - hi

    b0_hi, b0_lo = _hi_lo(lb0.reshape(H1, 1))
    b1_hi, b1_lo = _hi_lo(lb1.reshape(H2, 1))
    w0a = jnp.concatenate([lw0.T, b0_hi, b0_lo], axis=1)
    eye2 = jnp.concatenate(
        [jnp.zeros((2, C), jnp.float32), jnp.eye(2, dtype=jnp.float32)],
        axis=1)
    w0a = jnp.concatenate([w0a, eye2], axis=0)
    w0a = w0a.astype(jnp.bfloat16)
    w1a = jnp.concatenate([lw1.T, b1_hi, b1_lo],
                          axis=1).astype(jnp.bfloat16)

    means = pl.pallas_call(
        functools.partial(_point_mean_kernel, inv_l=1.0 / L, n_c=n_c),
        out_shape=jax.ShapeDtypeStruct((B, 1, H2), jnp.float32),
        grid=(B // bb,),
        in_specs=[
            pl.BlockSpec((bb, C, L), lambda i: (i, 0, 0)),
            pl.BlockSpec((H1 + 2, C + 2), lambda i: (0, 0)),
            pl.BlockSpec((H2, H1 + 2), lambda i: (0, 0)),
        ],
        out_specs=pl.BlockSpec((bb, 1, H2), lambda i: (i, 0, 0)),
        compiler_params=pltpu.CompilerParams(
            dimension_semantics=("arbitrary",),
            vmem_limit_bytes=100 * 1024 * 1024,
        ),
    )(point_cloud, w0a, w1a)

    m = means.reshape(B, H2)

    out = pl.pallas_call(
        _head_kernel,
        out_shape=jax.ShapeDtypeStruct((B, 1), jnp.float32),
    )(m, gw0, gb0.reshape(1, -1), gw1, gb1.reshape(1, -1),
      gw2, gb2.reshape(1, 1))
    return out


# speedup vs baseline: 1.4750x; 1.1351x over previous
import functools

import jax
import jax.numpy as jnp
from jax.experimental import pallas as pl
from jax.experimental.pallas import tpu as pltpu


def _point_mean_kernel(x_ref, w0_ref, w1_ref, o_ref, *, inv_l, n_c, wave=8):
    bb, _, L = x_ref.shape
    tc = L // n_c
    accs = [None] * bb
    lanes = [(b, c) for b in range(bb) for c in range(n_c)]
    g = min(wave, len(lanes))
    for w0 in range(0, len(lanes), g):
        wl = lanes[w0:w0 + g]
        h1s = []
        for b, c in wl:
            xb = x_ref[b, :, c * tc:(c + 1) * tc].astype(jnp.bfloat16)
            x_aug = jnp.concatenate(
                [xb, jnp.ones((2, tc), jnp.bfloat16)], axis=0)
            h1s.append(jnp.dot(w0_ref[...], x_aug,
                               preferred_element_type=jnp.float32))
        h1bs = [jnp.maximum(h1.astype(jnp.bfloat16), 0) for h1 in h1s]
        h2s = [jnp.dot(w1_ref[...], h1b, preferred_element_type=jnp.float32)
               for h1b in h1bs]
        h2s = [jnp.maximum(h2, 0.0) for h2 in h2s]
        for (b, c), h2 in zip(wl, h2s):
            w = tc
            while w > 128:
                w //= 2
                h2 = h2[:, :w] + h2[:, w:]
            accs[b] = h2 if accs[b] is None else accs[b] + h2
    for b in range(bb):
        ones = jnp.ones((1, 128), jnp.float32)
        row = jax.lax.dot_general(ones, accs[b] * inv_l,
                                  (((1,), (1,)), ((), ())),
                                  preferred_element_type=jnp.float32)
        o_ref[b] = row


def _head_kernel(m_ref, w0_ref, b0_ref, w1_ref, b1_ref, w2_ref, b2_ref, o_ref):
    g = jnp.dot(m_ref[...], w0_ref[...], preferred_element_type=jnp.float32)
    g = jnp.maximum(g + b0_ref[...], 0.0)
    g = jnp.dot(g, w1_ref[...], preferred_element_type=jnp.float32)
    g = jnp.maximum(g + b1_ref[...], 0.0)
    o_ref[...] = (jnp.dot(g, w2_ref[...], preferred_element_type=jnp.float32)
                  + b2_ref[...])


def _pick_tc(L):
    for tc in (512, 256, 128):
        if L % tc == 0:
            return tc
    return L


def kernel(point_cloud, lw0, lb0, lw1, lb1, gw0, gb0, gw1, gb1, gw2, gb2):
    B, C, L = point_cloud.shape
    H1 = lw0.shape[1]
    H2 = lw1.shape[1]

    tc = _pick_tc(L)
    n_c = L // tc
    bb = 8 if B % 8 == 0 else (2 if B % 2 == 0 else 1)

    def _hi_lo(b):
        hi = b.astype(jnp.bfloat16).astype(jnp.float32)
        return hi, b - hi

    b0_hi, b0_lo = _hi_lo(lb0.reshape(H1, 1))
    b1_hi, b1_lo = _hi_lo(lb1.reshape(H2, 1))
    w0a = jnp.concatenate([lw0.T, b0_hi, b0_lo], axis=1)
    eye2 = jnp.concatenate(
        [jnp.zeros((2, C), jnp.float32), jnp.eye(2, dtype=jnp.float32)],
        axis=1)
    w0a = jnp.concatenate([w0a, eye2], axis=0)
    w0a = w0a.astype(jnp.bfloat16)
    w1a = jnp.concatenate([lw1.T, b1_hi, b1_lo],
                          axis=1).astype(jnp.bfloat16)

    means = pl.pallas_call(
        functools.partial(_point_mean_kernel, inv_l=1.0 / L, n_c=n_c),
        out_shape=jax.ShapeDtypeStruct((B, 1, H2), jnp.float32),
        grid=(B // bb,),
        in_specs=[
            pl.BlockSpec((bb, C, L), lambda i: (i, 0, 0)),
            pl.BlockSpec((H1 + 2, C + 2), lambda i: (0, 0)),
            pl.BlockSpec((H2, H1 + 2), lambda i: (0, 0)),
        ],
        out_specs=pl.BlockSpec((bb, 1, H2), lambda i: (i, 0, 0)),
        compiler_params=pltpu.CompilerParams(
            dimension_semantics=("arbitrary",),
            vmem_limit_bytes=63 * 1024 * 1024,
        ),
    )(point_cloud, w0a, w1a)

    m = means.reshape(B, H2)

    out = pl.pallas_call(
        _head_kernel,
        out_shape=jax.ShapeDtypeStruct((B, 1), jnp.float32),
    )(m, gw0, gb0.reshape(1, -1), gw1, gb1.reshape(1, -1),
      gw2, gb2.reshape(1, 1))
    return out
